# Initial kernel scaffold; baseline (speedup 1.0000x reference)
import jax
import jax.numpy as jnp
from jax.experimental import pallas as pl


def kernel(x, edge_index, W_self0, W_neigh0, b0, W_self1, W_neigh1, b1, W_self2, W_neigh2, b2):
    raise NotImplementedError("write your pallas kernel here")


# speedup vs baseline: 4.5157x; 4.5157x over previous
import functools

import jax
import jax.numpy as jnp
from jax import lax
from jax.experimental import pallas as pl
from jax.experimental.pallas import tpu as pltpu
from jax.experimental.pallas import tpu_sc as plsc

N = 10000
E = 320000
D_IN = 128
D_H = 128
D_OUT = 64

NC = 2
NS = 16
NW = NC * NS

NP = 10240
RPS = NP // NS
CH = 128
EW = ((E // NW + CH - 1) // CH) * CH
EPAD = NW * EW
NCHUNK = EW // CH

R = 1280
GRID = NP // R


def _make_sc_agg(D, with_deg):
  mesh = plsc.VectorSubcoreMesh(
      core_axis_name="c", subcore_axis_name="s", num_cores=NC, num_subcores=NS)

  out_type = jax.ShapeDtypeStruct((NC, NP, D), jnp.float32)
  if with_deg:
    out_type = [out_type, jax.ShapeDtypeStruct((NC, NP), jnp.float32)]

  scratch = [
      pltpu.VMEM((CH,), jnp.int32),
      pltpu.VMEM((CH,), jnp.int32),
      pltpu.VMEM((CH, D), jnp.float32),
      pltpu.VMEM_SHARED((NP, D), jnp.float32),
      pltpu.SemaphoreType.DMA,
  ]
  if with_deg:
    scratch += [
        pltpu.VMEM((CH,), jnp.float32),
        pltpu.VMEM_SHARED((NP,), jnp.float32),
    ]

  def body(h_hbm, src_hbm, dst_hbm, z2_hbm, *rest):
    if with_deg:
      (z1_hbm, out_hbm, deg_hbm, src_v, dst_v, rows_v, acc_sh, sem,
       ones_v, deg_sh) = rest
    else:
      out_hbm, src_v, dst_v, rows_v, acc_sh, sem = rest

    c = lax.axis_index("c")
    s = lax.axis_index("s")
    w = s * NC + c

    pltpu.sync_copy(z2_hbm.at[pl.ds(s * RPS, RPS)],
                    acc_sh.at[pl.ds(s * RPS, RPS)])
    if with_deg:
      pltpu.sync_copy(z1_hbm.at[pl.ds(s * RPS, RPS)],
                      deg_sh.at[pl.ds(s * RPS, RPS)])
      for i in range(CH // 16):
        ones_v[pl.ds(i * 16, 16)] = jnp.ones((16,), jnp.float32)
    plsc.subcore_barrier()

    base = w * EW

    def chunk(j, carry):
      off = base + j * CH
      pltpu.sync_copy(src_hbm.at[pl.ds(off, CH)], src_v)
      pltpu.async_copy(h_hbm.at[src_v], rows_v, sem).wait()
      pltpu.sync_copy(dst_hbm.at[pl.ds(off, CH)], dst_v)
      pltpu.sync_copy(rows_v, acc_sh.at[dst_v], add=True)
      if with_deg:
        pltpu.sync_copy(ones_v, deg_sh.at[dst_v], add=True)
      return carry

    lax.fori_loop(0, NCHUNK, chunk, 0)
    plsc.subcore_barrier()

    pltpu.sync_copy(acc_sh.at[pl.ds(s * RPS, RPS)],
                    out_hbm.at[c, pl.ds(s * RPS, RPS)])
    if with_deg:
      pltpu.sync_copy(deg_sh.at[pl.ds(s * RPS, RPS)],
                      deg_hbm.at[c, pl.ds(s * RPS, RPS)])

  params = None
  if D % 128 != 0:
    params = pltpu.CompilerParams(use_tc_tiling_on_sc=False)
  return pl.kernel(body, out_type=out_type, mesh=mesh, scratch_types=scratch,
                   compiler_params=params,
                   name=f"sc_agg_d{D}" + ("_deg" if with_deg else ""))


_sc_agg_deg = _make_sc_agg(D_H, True)
_sc_agg = _make_sc_agg(D_H, False)
_sc_agg64 = _make_sc_agg(D_OUT, False)


def _dot(a, b):
  return jnp.dot(a, b, preferred_element_type=jnp.float32)


def _tc_layer0_body(h_ref, p_ref, d_ref, ws_ref, wn_ref, b_ref,
                    o_ref, invd_ref):
  invd = 1.0 / jnp.maximum(d_ref[0] + d_ref[1], 1.0)
  invd_ref[...] = invd
  agg = (p_ref[0] + p_ref[1]) * invd
  y = _dot(h_ref[...], ws_ref[...]) + _dot(agg, wn_ref[...]) + b_ref[...]
  o_ref[...] = jnp.maximum(y, 0.0)


def _tc_layer1_body(h_ref, p_ref, invd_ref, ws_ref, wn_ref, b_ref, wn2_ref,
                    o_ref, z_ref):
  agg = (p_ref[0] + p_ref[1]) * invd_ref[...]
  y = _dot(h_ref[...], ws_ref[...]) + _dot(agg, wn_ref[...]) + b_ref[...]
  h2 = jnp.maximum(y, 0.0)
  o_ref[...] = h2
  z_ref[...] = _dot(h2, wn2_ref[...])


def _tc_final_body(h_ref, p_ref, invd_ref, ws_ref, b_ref, o_ref):
  agg = (p_ref[0] + p_ref[1]) * invd_ref[...]
  o_ref[...] = _dot(h_ref[...], ws_ref[...]) + agg + b_ref[...]


def _row_block(d):
  return pl.BlockSpec((R, d), lambda i: (i, 0))


def _part_block(d):
  return pl.BlockSpec((NC, R, d), lambda i: (0, i, 0))


def _full(shape):
  return pl.BlockSpec(shape, lambda i: tuple(0 for _ in shape))


_tc_layer0 = pl.pallas_call(
    _tc_layer0_body,
    grid=(GRID,),
    in_specs=[_row_block(D_H), _part_block(D_H), _part_block(1),
              _full((D_IN, D_H)), _full((D_IN, D_H)), _full((1, D_H))],
    out_specs=[_row_block(D_H), _row_block(1)],
    out_shape=[jax.ShapeDtypeStruct((NP, D_H), jnp.float32),
               jax.ShapeDtypeStruct((NP, 1), jnp.float32)],
)

_tc_layer1 = pl.pallas_call(
    _tc_layer1_body,
    grid=(GRID,),
    in_specs=[_row_block(D_H), _part_block(D_H), _row_block(1),
              _full((D_H, D_H)), _full((D_H, D_H)), _full((1, D_H)),
              _full((D_H, D_OUT))],
    out_specs=[_row_block(D_H), _row_block(D_OUT)],
    out_shape=[jax.ShapeDtypeStruct((NP, D_H), jnp.float32),
               jax.ShapeDtypeStruct((NP, D_OUT), jnp.float32)],
)

_tc_final = pl.pallas_call(
    _tc_final_body,
    grid=(GRID,),
    in_specs=[_row_block(D_H), _part_block(D_OUT), _row_block(1),
              _full((D_H, D_OUT)), _full((1, D_OUT))],
    out_specs=_row_block(D_OUT),
    out_shape=jax.ShapeDtypeStruct((NP, D_OUT), jnp.float32),
)


def kernel(x, edge_index, W_self0, W_neigh0, b0, W_self1, W_neigh1, b1,
           W_self2, W_neigh2, b2):
  src = edge_index[0]
  dst = edge_index[1]
  src_p = jnp.concatenate(
      [src, jnp.zeros((EPAD - E,), jnp.int32)])
  dst_p = jnp.concatenate(
      [dst, jnp.full((EPAD - E,), N, jnp.int32)])

  h0 = jnp.pad(x, ((0, NP - N), (0, 0)))
  z2d = jnp.zeros((NP, D_H), jnp.float32)
  z1d = jnp.zeros((NP,), jnp.float32)

  p0, degp = _sc_agg_deg(h0, src_p, dst_p, z2d, z1d)
  h1, invd = _tc_layer0(h0, p0, degp[..., None], W_self0, W_neigh0,
                        b0.reshape(1, D_H))
  p1 = _sc_agg(h1, src_p, dst_p, z2d)
  h2, z2 = _tc_layer1(h1, p1, invd, W_self1, W_neigh1, b1.reshape(1, D_H),
                      W_neigh2)
  pz = _sc_agg64(z2, src_p, dst_p, z2d[:, :D_OUT])
  out = _tc_final(h2, pz, invd, W_self2, b2.reshape(1, D_OUT))
  return out[:N]
